# baseline (speedup 1.0000x reference)
import functools

import jax
import jax.numpy as jnp
from jax import lax
from jax.experimental import pallas as pl
from jax.experimental.pallas import tpu as pltpu

C = 8
B = 4096
D = 768
BB = 512

_DNT = (((1,), (1,)), ((), ()))


def _sigmoid(x):
    return 0.5 * jnp.tanh(0.5 * x) + 0.5


def _cell_kernel(cm_ref, ch_ref, wf_ref, bf_ref, wiou_ref, biou_ref,
                 nm_ref, nh_ref):
    wf = wf_ref[...]
    bfv = bf_ref[...]
    hs = None
    fsum = None
    for c in range(C):
        hc = ch_ref[c]
        hs = hc if hs is None else hs + hc
        fl = lax.dot_general(hc.astype(jnp.bfloat16), wf, _DNT,
                             preferred_element_type=jnp.float32) + bfv
        fm = _sigmoid(fl) * cm_ref[c]
        fsum = fm if fsum is None else fsum + fm

    iou = lax.dot_general(hs.astype(jnp.bfloat16), wiou_ref[...], _DNT,
                          preferred_element_type=jnp.float32) + biou_ref[...]
    input_gate = _sigmoid(iou[:, :D])
    output_gate = _sigmoid(iou[:, D:2 * D])
    memory_gate = jnp.tanh(iou[:, 2 * D:])

    nm = input_gate * memory_gate + fsum
    nm_ref[...] = nm
    nh_ref[...] = output_gate * jnp.tanh(nm)


@functools.partial(jax.jit, static_argnames=("interpret",))
def kernel(children_memory, children_hidden, Wf, bf, Wiou, biou,
           interpret=False):
    wf16 = Wf.astype(jnp.bfloat16)
    wiou16 = Wiou.astype(jnp.bfloat16)
    bf2 = bf.reshape(1, D)
    biou2 = biou.reshape(1, 3 * D)

    grid = (B // BB,)
    nm, nh = pl.pallas_call(
        _cell_kernel,
        grid=grid,
        in_specs=[
            pl.BlockSpec((C, BB, D), lambda i: (0, i, 0)),
            pl.BlockSpec((C, BB, D), lambda i: (0, i, 0)),
            pl.BlockSpec((D, D), lambda i: (0, 0)),
            pl.BlockSpec((1, D), lambda i: (0, 0)),
            pl.BlockSpec((3 * D, D), lambda i: (0, 0)),
            pl.BlockSpec((1, 3 * D), lambda i: (0, 0)),
        ],
        out_specs=[
            pl.BlockSpec((BB, D), lambda i: (i, 0)),
            pl.BlockSpec((BB, D), lambda i: (i, 0)),
        ],
        out_shape=[
            jax.ShapeDtypeStruct((B, D), jnp.float32),
            jax.ShapeDtypeStruct((B, D), jnp.float32),
        ],
        compiler_params=pltpu.CompilerParams(
            dimension_semantics=("parallel",),
            vmem_limit_bytes=117440512,
        ),
        interpret=interpret,
    )(children_memory, children_hidden, wf16, bf2, wiou16, biou2)
    return (nm, nh)


# speedup vs baseline: 1.1804x; 1.1804x over previous
import functools

import jax
import jax.numpy as jnp
from jax import lax
from jax.experimental import pallas as pl
from jax.experimental.pallas import tpu as pltpu

C = 8
B = 4096
D = 768
BB = 256
NSTEP = B // BB
NBUF = 3

_DNT = (((1,), (1,)), ((), ()))


def _sigmoid(x):
    return 0.5 * jnp.tanh(0.5 * x) + 0.5


def _pipe_kernel(cm_hbm, ch_hbm, wf_ref, bf_ref, wiou_ref, biou_ref,
                 nm_hbm, nh_hbm,
                 cm_buf, ch_buf, nm_buf, nh_buf, in_sems, out_sems):
    def in_copies(i, slot):
        return (
            pltpu.make_async_copy(cm_hbm.at[:, pl.ds(i * BB, BB), :],
                                  cm_buf.at[slot], in_sems.at[slot, 0]),
            pltpu.make_async_copy(ch_hbm.at[:, pl.ds(i * BB, BB), :],
                                  ch_buf.at[slot], in_sems.at[slot, 1]),
        )

    def out_copies(i, oslot):
        return (
            pltpu.make_async_copy(nm_buf.at[oslot],
                                  nm_hbm.at[pl.ds(i * BB, BB), :],
                                  out_sems.at[oslot, 0]),
            pltpu.make_async_copy(nh_buf.at[oslot],
                                  nh_hbm.at[pl.ds(i * BB, BB), :],
                                  out_sems.at[oslot, 1]),
        )

    for i in range(NBUF):
        for cpy in in_copies(i, i):
            cpy.start()

    wf = wf_ref[...].astype(jnp.bfloat16)
    bfv = bf_ref[...]
    wiou = wiou_ref[...].astype(jnp.bfloat16)
    biouv = biou_ref[...]

    def body(i, carry):
        slot = lax.rem(i, NBUF)
        oslot = lax.rem(i, 2)
        for cpy in in_copies(i, slot):
            cpy.wait()

        @pl.when(i + NBUF < NSTEP)
        def _():
            for cpy in in_copies(i + NBUF, lax.rem(i + NBUF, NBUF)):
                cpy.start()

        @pl.when(i >= 2)
        def _():
            for cpy in out_copies(i - 2, oslot):
                cpy.wait()

        hs = None
        fsum = None
        for c in range(C):
            hc = ch_buf[slot, c]
            hs = hc if hs is None else hs + hc
            fl = lax.dot_general(hc.astype(jnp.bfloat16), wf, _DNT,
                                 preferred_element_type=jnp.float32) + bfv
            fm = _sigmoid(fl) * cm_buf[slot, c]
            fsum = fm if fsum is None else fsum + fm

        iou = lax.dot_general(hs.astype(jnp.bfloat16), wiou, _DNT,
                              preferred_element_type=jnp.float32) + biouv
        input_gate = _sigmoid(iou[:, :D])
        output_gate = _sigmoid(iou[:, D:2 * D])
        memory_gate = jnp.tanh(iou[:, 2 * D:])

        nm = input_gate * memory_gate + fsum
        nm_buf[oslot] = nm
        nh_buf[oslot] = output_gate * jnp.tanh(nm)

        for cpy in out_copies(i, oslot):
            cpy.start()
        return carry

    lax.fori_loop(0, NSTEP, body, 0)

    for i in (NSTEP - 2, NSTEP - 1):
        for cpy in out_copies(i, i % 2):
            cpy.wait()


@functools.partial(jax.jit, static_argnames=("interpret",))
def kernel(children_memory, children_hidden, Wf, bf, Wiou, biou,
           interpret=False):
    bf2 = bf.reshape(1, D)
    biou2 = biou.reshape(1, 3 * D)

    nm, nh = pl.pallas_call(
        _pipe_kernel,
        in_specs=[
            pl.BlockSpec(memory_space=pl.ANY),
            pl.BlockSpec(memory_space=pl.ANY),
            pl.BlockSpec((D, D), lambda: (0, 0)),
            pl.BlockSpec((1, D), lambda: (0, 0)),
            pl.BlockSpec((3 * D, D), lambda: (0, 0)),
            pl.BlockSpec((1, 3 * D), lambda: (0, 0)),
        ],
        out_specs=[
            pl.BlockSpec(memory_space=pl.ANY),
            pl.BlockSpec(memory_space=pl.ANY),
        ],
        out_shape=[
            jax.ShapeDtypeStruct((B, D), jnp.float32),
            jax.ShapeDtypeStruct((B, D), jnp.float32),
        ],
        scratch_shapes=[
            pltpu.VMEM((NBUF, C, BB, D), jnp.float32),
            pltpu.VMEM((NBUF, C, BB, D), jnp.float32),
            pltpu.VMEM((2, BB, D), jnp.float32),
            pltpu.VMEM((2, BB, D), jnp.float32),
            pltpu.SemaphoreType.DMA((NBUF, 2)),
            pltpu.SemaphoreType.DMA((2, 2)),
        ],
        interpret=interpret,
    )(children_memory, children_hidden, Wf, bf2, Wiou, biou2)
    return (nm, nh)
